# initial kernel scaffold (unmeasured)
import jax
import jax.numpy as jnp
from jax import lax
from jax.experimental import pallas as pl
from jax.experimental.pallas import tpu as pltpu

N_DEV = 4
S_LOC = 512
S_GLOB = N_DEV * S_LOC
D = 1024
HL = 8
DH = 128
SCALE = 0.08838834764831843

BF16 = jnp.bfloat16
F32 = jnp.float32


def kernel(x, Wq, Wo, Wk, Wv):
    x2 = x.reshape(S_LOC, D).astype(BF16)
    wq = Wq.astype(BF16)
    wk = Wk.astype(BF16)
    wv = Wv.astype(BF16)
    wo = Wo.astype(BF16)

    def body(x_ref, wq_ref, wo_ref, wk_ref, wv_ref, out_ref,
             xg, ao, pbuf, sb, rb, ag_send, ag_recv, rs_send, rs_recv):
        my = lax.axis_index("i")
        left = lax.rem(my + N_DEV - 1, N_DEV)
        right = lax.rem(my + 1, N_DEV)

        barrier = pltpu.get_barrier_semaphore()
        for nbr in (left, right):
            pl.semaphore_signal(
                barrier, inc=1,
                device_id=(nbr,), device_id_type=pl.DeviceIdType.MESH,
            )
        pl.semaphore_wait(barrier, 2)

        xg[pl.ds(my * S_LOC, S_LOC), :] = x_ref[...]
        for h in range(N_DEV - 1):
            o = lax.rem(my + N_DEV - h, N_DEV)
            rdma = pltpu.make_async_remote_copy(
                src_ref=xg.at[pl.ds(o * S_LOC, S_LOC), :],
                dst_ref=xg.at[pl.ds(o * S_LOC, S_LOC), :],
                send_sem=ag_send.at[h],
                recv_sem=ag_recv.at[h],
                device_id=(right,),
                device_id_type=pl.DeviceIdType.MESH,
            )
            rdma.start()
            rdma.wait()

        xv = xg[...]
        q_all = jnp.dot(xv, wq_ref[...], preferred_element_type=F32).astype(BF16)
        k_all = jnp.dot(xv, wk_ref[...], preferred_element_type=F32).astype(BF16)
        v_all = jnp.dot(xv, wv_ref[...], preferred_element_type=F32).astype(BF16)
        for h in range(HL):
            sl = slice(h * DH, (h + 1) * DH)
            qh = q_all[:, sl]
            kh = k_all[:, sl]
            vh = v_all[:, sl]
            s = lax.dot_general(
                qh, kh, (((1,), (1,)), ((), ())), preferred_element_type=F32
            ) * SCALE
            m = jnp.max(s, axis=-1, keepdims=True)
            p = jnp.exp(s - m)
            l = jnp.sum(p, axis=-1, keepdims=True)
            o_h = lax.dot_general(
                p.astype(BF16), vh, (((1,), (0,)), ((), ())),
                preferred_element_type=F32,
            )
            ao[:, sl] = o_h / l

        pbuf[...] = jnp.dot(
            ao[...].astype(BF16), wo_ref[...], preferred_element_type=F32
        ).astype(BF16)

        c0 = lax.rem(my + N_DEV - 1, N_DEV)
        sb[...] = pbuf[pl.ds(c0 * S_LOC, S_LOC), :]
        for s in range(N_DEV - 1):
            rdma = pltpu.make_async_remote_copy(
                src_ref=sb,
                dst_ref=rb.at[s],
                send_sem=rs_send.at[s],
                recv_sem=rs_recv.at[s],
                device_id=(right,),
                device_id_type=pl.DeviceIdType.MESH,
            )
            rdma.start()
            rdma.wait()
            cr = lax.rem(my + 2 * N_DEV - 2 - s, N_DEV)
            acc = (rb[s].astype(F32)
                   + pbuf[pl.ds(cr * S_LOC, S_LOC), :].astype(F32))
            if s < N_DEV - 2:
                sb[...] = acc.astype(BF16)
            else:
                out_ref[...] = acc

    out = pl.pallas_call(
        body,
        out_shape=jax.ShapeDtypeStruct((S_LOC, D), F32),
        in_specs=[pl.BlockSpec(memory_space=pltpu.VMEM)] * 5,
        out_specs=pl.BlockSpec(memory_space=pltpu.VMEM),
        scratch_shapes=[
            pltpu.VMEM((S_GLOB, D), BF16),
            pltpu.VMEM((S_GLOB, D), F32),
            pltpu.VMEM((S_GLOB, D), BF16),
            pltpu.VMEM((S_LOC, D), BF16),
            pltpu.VMEM((N_DEV - 1, S_LOC, D), BF16),
            pltpu.SemaphoreType.DMA((N_DEV - 1,)),
            pltpu.SemaphoreType.DMA((N_DEV - 1,)),
            pltpu.SemaphoreType.DMA((N_DEV - 1,)),
            pltpu.SemaphoreType.DMA((N_DEV - 1,)),
        ],
        compiler_params=pltpu.CompilerParams(collective_id=0),
    )(x2, wq, wo, wk, wv)
    return out.reshape(1, S_LOC, D)


# baseline (device time: 228391 ns/iter reference)
import jax
import jax.numpy as jnp
from jax import lax
from jax.experimental import pallas as pl
from jax.experimental.pallas import tpu as pltpu

N_DEV = 4
S_LOC = 512
S_GLOB = N_DEV * S_LOC
D = 1024
HL = 8
DH = 128
SCALE = 0.08838834764831843

BF16 = jnp.bfloat16
F32 = jnp.float32


def kernel(x, Wq, Wo, Wk, Wv):
    x2 = x.reshape(S_LOC, D).astype(BF16)
    wq = Wq.astype(BF16)
    wk = Wk.astype(BF16)
    wv = Wv.astype(BF16)
    wo = Wo.astype(BF16)

    def body(x_ref, wq_ref, wo_ref, wk_ref, wv_ref, out_ref,
             xg, khb, vhb, pbuf, sb, rb, ag_send, ag_recv, rs_send, rs_recv):
        my = lax.axis_index("i")
        left = lax.rem(my + N_DEV - 1, N_DEV)
        right = lax.rem(my + 1, N_DEV)

        barrier = pltpu.get_barrier_semaphore()
        for nbr in (left, right):
            pl.semaphore_signal(
                barrier, inc=1,
                device_id=(nbr,), device_id_type=pl.DeviceIdType.MESH,
            )
        pl.semaphore_wait(barrier, 2)

        xg[pl.ds(my * S_LOC, S_LOC), :] = x_ref[...]
        for h in range(N_DEV - 1):
            o = lax.rem(my + N_DEV - h, N_DEV)
            rdma = pltpu.make_async_remote_copy(
                src_ref=xg.at[pl.ds(o * S_LOC, S_LOC), :],
                dst_ref=xg.at[pl.ds(o * S_LOC, S_LOC), :],
                send_sem=ag_send.at[h],
                recv_sem=ag_recv.at[h],
                device_id=(right,),
                device_id_type=pl.DeviceIdType.MESH,
            )
            rdma.start()
            rdma.wait()

        pbuf[...] = jnp.zeros((S_GLOB, D), F32)

        def head_body(h, _):
            hs = pl.ds(h * DH, DH)
            khb[...] = jnp.dot(xg[...], wk_ref[:, hs],
                               preferred_element_type=F32).astype(BF16)
            vhb[...] = jnp.dot(xg[...], wv_ref[:, hs],
                               preferred_element_type=F32).astype(BF16)

            def qb_body(qb, _):
                rows = pl.ds(qb * S_LOC, S_LOC)
                qh = jnp.dot(xg[rows, :], wq_ref[:, hs],
                             preferred_element_type=F32).astype(BF16)
                s = lax.dot_general(
                    qh, khb[...], (((1,), (1,)), ((), ())),
                    preferred_element_type=F32,
                ) * SCALE
                m = jnp.max(s, axis=-1, keepdims=True)
                p = jnp.exp(s - m)
                l = jnp.sum(p, axis=-1, keepdims=True)
                o_h = lax.dot_general(
                    p.astype(BF16), vhb[...], (((1,), (0,)), ((), ())),
                    preferred_element_type=F32,
                )
                pbuf[rows, :] += jnp.dot(
                    (o_h / l).astype(BF16), wo_ref[hs, :],
                    preferred_element_type=F32,
                )
                return 0

            lax.fori_loop(0, N_DEV, qb_body, 0)
            return 0

        lax.fori_loop(0, HL, head_body, 0)

        c0 = lax.rem(my + N_DEV - 1, N_DEV)
        sb[...] = pbuf[pl.ds(c0 * S_LOC, S_LOC), :].astype(BF16)
        for s in range(N_DEV - 1):
            rdma = pltpu.make_async_remote_copy(
                src_ref=sb,
                dst_ref=rb.at[s],
                send_sem=rs_send.at[s],
                recv_sem=rs_recv.at[s],
                device_id=(right,),
                device_id_type=pl.DeviceIdType.MESH,
            )
            rdma.start()
            rdma.wait()
            cr = lax.rem(my + 2 * N_DEV - 2 - s, N_DEV)
            acc = rb[s].astype(F32) + pbuf[pl.ds(cr * S_LOC, S_LOC), :]
            if s < N_DEV - 2:
                sb[...] = acc.astype(BF16)
            else:
                out_ref[...] = acc

    out = pl.pallas_call(
        body,
        out_shape=jax.ShapeDtypeStruct((S_LOC, D), F32),
        in_specs=[pl.BlockSpec(memory_space=pltpu.VMEM)] * 5,
        out_specs=pl.BlockSpec(memory_space=pltpu.VMEM),
        scratch_shapes=[
            pltpu.VMEM((S_GLOB, D), BF16),
            pltpu.VMEM((S_GLOB, DH), BF16),
            pltpu.VMEM((S_GLOB, DH), BF16),
            pltpu.VMEM((S_GLOB, D), F32),
            pltpu.VMEM((S_LOC, D), BF16),
            pltpu.VMEM((N_DEV - 1, S_LOC, D), BF16),
            pltpu.SemaphoreType.DMA((N_DEV - 1,)),
            pltpu.SemaphoreType.DMA((N_DEV - 1,)),
            pltpu.SemaphoreType.DMA((N_DEV - 1,)),
            pltpu.SemaphoreType.DMA((N_DEV - 1,)),
        ],
        compiler_params=pltpu.CompilerParams(collective_id=0),
    )(x2, wq, wo, wk, wv)
    return out.reshape(1, S_LOC, D)


# device time: 120557 ns/iter; 1.8945x vs baseline; 1.8945x over previous
import jax
import jax.numpy as jnp
from jax import lax
from jax.experimental import pallas as pl
from jax.experimental.pallas import tpu as pltpu

N_DEV = 4
S_LOC = 512
S_GLOB = N_DEV * S_LOC
D = 1024
HL = 8
DH = 128
SCALE = 0.08838834764831843

BF16 = jnp.bfloat16
F32 = jnp.float32
MESHID = pl.DeviceIdType.MESH


def kernel(x, Wq, Wo, Wk, Wv):
    x2 = x.reshape(S_LOC, D).astype(BF16)
    wq = Wq.astype(BF16)
    wk = Wk.astype(BF16)
    wv = Wv.astype(BF16)
    wo = Wo.astype(BF16)

    def body(x_ref, wq_ref, wo_ref, wk_ref, wv_ref, out_ref,
             xg, kf, vf, qfr, accr, sbuf, rbuf,
             ag_send, ag_recv, rs_send, rs_recv):
        my = lax.axis_index("i")

        barrier = pltpu.get_barrier_semaphore()
        for d in (1, 2, 3):
            pl.semaphore_signal(
                barrier, inc=1,
                device_id=(lax.rem(my + d, N_DEV),), device_id_type=MESHID,
            )
        pl.semaphore_wait(barrier, 3)

        my_rows = pl.ds(my * S_LOC, S_LOC)
        xg[my_rows, :] = x_ref[...]
        sends = []
        for d in (1, 2, 3):
            rd = pltpu.make_async_remote_copy(
                src_ref=xg.at[my_rows, :],
                dst_ref=xg.at[my_rows, :],
                send_sem=ag_send.at[d - 1],
                recv_sem=ag_recv.at[d - 1],
                device_id=(lax.rem(my + d, N_DEV),),
                device_id_type=MESHID,
            )
            rd.start()
            sends.append(rd)

        kf[my_rows, :] = jnp.dot(xg[my_rows, :], wk_ref[...],
                                 preferred_element_type=F32).astype(BF16)
        vf[my_rows, :] = jnp.dot(xg[my_rows, :], wv_ref[...],
                                 preferred_element_type=F32).astype(BF16)

        for d in (1, 3, 2):
            o_rows = pl.ds(lax.rem(my + N_DEV - d, N_DEV) * S_LOC, S_LOC)
            rcv = pltpu.make_async_remote_copy(
                src_ref=xg.at[o_rows, :],
                dst_ref=xg.at[o_rows, :],
                send_sem=ag_send.at[d - 1],
                recv_sem=ag_recv.at[d - 1],
                device_id=(my,), device_id_type=MESHID,
            )
            rcv.wait_recv()
            kf[o_rows, :] = jnp.dot(xg[o_rows, :], wk_ref[...],
                                    preferred_element_type=F32).astype(BF16)
            vf[o_rows, :] = jnp.dot(xg[o_rows, :], wv_ref[...],
                                    preferred_element_type=F32).astype(BF16)

        def attn_block(rows):
            accr[...] = jnp.zeros((S_LOC, D), F32)
            qfr[...] = jnp.dot(xg[rows, :], wq_ref[...],
                               preferred_element_type=F32).astype(BF16)

            def hbody(h, _):
                hs = pl.ds(h * DH, DH)
                qh = qfr[:, hs]
                s = lax.dot_general(
                    qh, kf[:, hs], (((1,), (1,)), ((), ())),
                    preferred_element_type=F32,
                ) * SCALE
                p = jnp.exp(s)
                l = jnp.sum(p, axis=-1, keepdims=True)
                ob = lax.dot_general(
                    p.astype(BF16), vf[:, hs], (((1,), (0,)), ((), ())),
                    preferred_element_type=F32,
                )
                accr[...] += jnp.dot((ob / l).astype(BF16), wo_ref[hs, :],
                                     preferred_element_type=F32)
                return 0

            lax.fori_loop(0, HL, hbody, 0)

        for d in (2, 1, 3):
            tgt = lax.rem(my + d, N_DEV)
            attn_block(pl.ds(tgt * S_LOC, S_LOC))
            sbuf[d - 1, :, :] = accr[...].astype(BF16)
            rd = pltpu.make_async_remote_copy(
                src_ref=sbuf.at[d - 1],
                dst_ref=rbuf.at[d - 1],
                send_sem=rs_send.at[d - 1],
                recv_sem=rs_recv.at[d - 1],
                device_id=(tgt,), device_id_type=MESHID,
            )
            rd.start()
            sends.append(rd)

        attn_block(my_rows)

        for d in (1, 3, 2):
            rcv = pltpu.make_async_remote_copy(
                src_ref=sbuf.at[d - 1],
                dst_ref=rbuf.at[d - 1],
                send_sem=rs_send.at[d - 1],
                recv_sem=rs_recv.at[d - 1],
                device_id=(my,), device_id_type=MESHID,
            )
            rcv.wait_recv()
        out_ref[...] = (accr[...]
                        + rbuf[0].astype(F32)
                        + rbuf[1].astype(F32)
                        + rbuf[2].astype(F32))

        for rd in sends:
            rd.wait_send()

    out = pl.pallas_call(
        body,
        out_shape=jax.ShapeDtypeStruct((S_LOC, D), F32),
        in_specs=[pl.BlockSpec(memory_space=pltpu.VMEM)] * 5,
        out_specs=pl.BlockSpec(memory_space=pltpu.VMEM),
        scratch_shapes=[
            pltpu.VMEM((S_GLOB, D), BF16),
            pltpu.VMEM((S_GLOB, D), BF16),
            pltpu.VMEM((S_GLOB, D), BF16),
            pltpu.VMEM((S_LOC, D), BF16),
            pltpu.VMEM((S_LOC, D), F32),
            pltpu.VMEM((N_DEV - 1, S_LOC, D), BF16),
            pltpu.VMEM((N_DEV - 1, S_LOC, D), BF16),
            pltpu.SemaphoreType.DMA((N_DEV - 1,)),
            pltpu.SemaphoreType.DMA((N_DEV - 1,)),
            pltpu.SemaphoreType.DMA((N_DEV - 1,)),
            pltpu.SemaphoreType.DMA((N_DEV - 1,)),
        ],
        compiler_params=pltpu.CompilerParams(collective_id=0),
    )(x2, wq, wo, wk, wv)
    return out.reshape(1, S_LOC, D)


# device time: 100596 ns/iter; 2.2704x vs baseline; 1.1984x over previous
import jax
import jax.numpy as jnp
from jax import lax
from jax.experimental import pallas as pl
from jax.experimental.pallas import tpu as pltpu

N_DEV = 4
S_LOC = 512
S_GLOB = N_DEV * S_LOC
D = 1024
HL = 8
DH = 128
SCALE = 0.08838834764831843

BF16 = jnp.bfloat16
F32 = jnp.float32
MESHID = pl.DeviceIdType.MESH


def kernel(x, Wq, Wo, Wk, Wv):
    x2 = x.reshape(S_LOC, D).astype(BF16)
    wq = Wq.astype(BF16)
    wk = Wk.astype(BF16)
    wv = Wv.astype(BF16)
    wo = Wo.astype(BF16)

    def body(x_ref, wq_ref, wo_ref, wk_ref, wv_ref, out_ref,
             xg, kf, vf, qfr, obr, sbuf, rbuf,
             ag_send, ag_recv, rs_send, rs_recv):
        my = lax.axis_index("i")

        barrier = pltpu.get_barrier_semaphore()
        for d in (1, 2, 3):
            pl.semaphore_signal(
                barrier, inc=1,
                device_id=(lax.rem(my + d, N_DEV),), device_id_type=MESHID,
            )
        pl.semaphore_wait(barrier, 3)

        my_rows = pl.ds(my * S_LOC, S_LOC)
        xg[my_rows, :] = x_ref[...]
        sends = []
        for d in (1, 2, 3):
            rd = pltpu.make_async_remote_copy(
                src_ref=xg.at[my_rows, :],
                dst_ref=xg.at[my_rows, :],
                send_sem=ag_send.at[d - 1],
                recv_sem=ag_recv.at[d - 1],
                device_id=(lax.rem(my + d, N_DEV),),
                device_id_type=MESHID,
            )
            rd.start()
            sends.append(rd)

        kf[my_rows, :] = jnp.dot(xg[my_rows, :], wk_ref[...],
                                 preferred_element_type=F32).astype(BF16)
        vf[my_rows, :] = jnp.dot(xg[my_rows, :], wv_ref[...],
                                 preferred_element_type=F32).astype(BF16)

        for d in (1, 3, 2):
            o_rows = pl.ds(lax.rem(my + N_DEV - d, N_DEV) * S_LOC, S_LOC)
            rcv = pltpu.make_async_remote_copy(
                src_ref=xg.at[o_rows, :],
                dst_ref=xg.at[o_rows, :],
                send_sem=ag_send.at[d - 1],
                recv_sem=ag_recv.at[d - 1],
                device_id=(my,), device_id_type=MESHID,
            )
            rcv.wait_recv()
            kf[o_rows, :] = jnp.dot(xg[o_rows, :], wk_ref[...],
                                    preferred_element_type=F32).astype(BF16)
            vf[o_rows, :] = jnp.dot(xg[o_rows, :], wv_ref[...],
                                    preferred_element_type=F32).astype(BF16)

        def attn_block(rows):
            qfr[...] = jnp.dot(xg[rows, :], wq_ref[...],
                               preferred_element_type=F32).astype(BF16)

            def hbody(h, _):
                hs = pl.ds(h * DH, DH)
                qh = qfr[:, hs]
                s = lax.dot_general(
                    qh, kf[:, hs], (((1,), (1,)), ((), ())),
                    preferred_element_type=F32,
                ) * SCALE
                p = jnp.exp(s)
                l = jnp.sum(p, axis=-1, keepdims=True)
                ob = lax.dot_general(
                    p.astype(BF16), vf[:, hs], (((1,), (0,)), ((), ())),
                    preferred_element_type=F32,
                )
                obr[:, hs] = (ob / l).astype(BF16)
                return 0

            lax.fori_loop(0, HL, hbody, 0)
            return jnp.dot(obr[...], wo_ref[...], preferred_element_type=F32)

        for d in (2, 1, 3):
            tgt = lax.rem(my + d, N_DEV)
            sbuf[d - 1, :, :] = attn_block(
                pl.ds(tgt * S_LOC, S_LOC)).astype(BF16)
            rd = pltpu.make_async_remote_copy(
                src_ref=sbuf.at[d - 1],
                dst_ref=rbuf.at[d - 1],
                send_sem=rs_send.at[d - 1],
                recv_sem=rs_recv.at[d - 1],
                device_id=(tgt,), device_id_type=MESHID,
            )
            rd.start()
            sends.append(rd)

        own = attn_block(my_rows)

        for d in (1, 3, 2):
            rcv = pltpu.make_async_remote_copy(
                src_ref=sbuf.at[d - 1],
                dst_ref=rbuf.at[d - 1],
                send_sem=rs_send.at[d - 1],
                recv_sem=rs_recv.at[d - 1],
                device_id=(my,), device_id_type=MESHID,
            )
            rcv.wait_recv()
        out_ref[...] = (own
                        + rbuf[0].astype(F32)
                        + rbuf[1].astype(F32)
                        + rbuf[2].astype(F32))

        for rd in sends:
            rd.wait_send()

    out = pl.pallas_call(
        body,
        out_shape=jax.ShapeDtypeStruct((S_LOC, D), F32),
        in_specs=[pl.BlockSpec(memory_space=pltpu.VMEM)] * 5,
        out_specs=pl.BlockSpec(memory_space=pltpu.VMEM),
        scratch_shapes=[
            pltpu.VMEM((S_GLOB, D), BF16),
            pltpu.VMEM((S_GLOB, D), BF16),
            pltpu.VMEM((S_GLOB, D), BF16),
            pltpu.VMEM((S_LOC, D), BF16),
            pltpu.VMEM((S_LOC, D), BF16),
            pltpu.VMEM((N_DEV - 1, S_LOC, D), BF16),
            pltpu.VMEM((N_DEV - 1, S_LOC, D), BF16),
            pltpu.SemaphoreType.DMA((N_DEV - 1,)),
            pltpu.SemaphoreType.DMA((N_DEV - 1,)),
            pltpu.SemaphoreType.DMA((N_DEV - 1,)),
            pltpu.SemaphoreType.DMA((N_DEV - 1,)),
        ],
        compiler_params=pltpu.CompilerParams(collective_id=0),
    )(x2, wq, wo, wk, wv)
    return out.reshape(1, S_LOC, D)
